# Initial kernel scaffold; baseline (speedup 1.0000x reference)
import jax
import jax.numpy as jnp
from jax.experimental import pallas as pl


def kernel(z, codebook):
    raise NotImplementedError("write your pallas kernel here")


# speedup vs baseline: 1.1327x; 1.1327x over previous
import functools

import jax
import jax.numpy as jnp
from jax import lax
from jax.experimental import pallas as pl
from jax.experimental.pallas import tpu as pltpu
from jax.experimental.pallas import tpu_sc as plsc

EMBED = 256
NCODES = 8192
ROWS = 8 * 576

_SC_CORES = 2
_SC_SUBCORES = 16
_SC_WORKERS = _SC_CORES * _SC_SUBCORES
_ROWS_PER_WORKER = ROWS // _SC_WORKERS

_TM = 256
_GRID = ROWS // _TM
_TM2 = 512
_GRID2 = ROWS // _TM2
_EPS = 1e-07


def _normalize(x, axis):
    n = jnp.sqrt(jnp.sum(x * x, axis=axis, keepdims=True))
    return x / jnp.maximum(n, _EPS)


def _distance_body(z_ref, cbt_ref, idx_ref, et_ref, ne_ref):
    i = pl.program_id(0)

    @pl.when(i == 0)
    def _init():
        cbt = cbt_ref[...]
        et = _normalize(cbt, axis=0)
        et_ref[...] = et.astype(jnp.bfloat16)
        ne_ref[...] = jnp.sum(et * et, axis=0, keepdims=True)

    zt = z_ref[...]
    zn = _normalize(zt, axis=1)
    sumz = jnp.sum(zn * zn, axis=1, keepdims=True)
    lhs = (2.0 * zn).astype(jnp.bfloat16)
    s2 = jnp.dot(lhs, et_ref[...],
                 preferred_element_type=jnp.float32)
    d = (sumz - s2) + ne_ref[...]
    m = jnp.min(d, axis=1, keepdims=True)
    col = lax.broadcasted_iota(jnp.int32, d.shape, 1)
    idx = jnp.min(jnp.where(d == m, col, jnp.int32(NCODES)), axis=1)
    idx_ref[0, 0, :] = idx


def _distance_indices(z_flat, cb_t):
    return pl.pallas_call(
        _distance_body,
        grid=(_GRID,),
        in_specs=[
            pl.BlockSpec((_TM, EMBED), lambda i: (i, 0)),
            pl.BlockSpec((EMBED, NCODES), lambda i: (0, 0)),
        ],
        out_specs=pl.BlockSpec((1, 1, _TM), lambda i: (i, 0, 0)),
        out_shape=jax.ShapeDtypeStruct((_GRID, 1, _TM), jnp.int32),
        scratch_shapes=[
            pltpu.VMEM((EMBED, NCODES), jnp.bfloat16),
            pltpu.VMEM((1, NCODES), jnp.float32),
        ],
    )(z_flat, cb_t)


def _sc_gather_body(cb_hbm, idx_hbm, out_hbm, idx_v, rows_v, sem):
    wid = lax.axis_index("s") * _SC_CORES + lax.axis_index("c")
    base = wid * _ROWS_PER_WORKER
    pltpu.sync_copy(idx_hbm.at[pl.ds(base, _ROWS_PER_WORKER)], idx_v)
    pltpu.async_copy(cb_hbm.at[idx_v], rows_v, sem).wait()
    pltpu.sync_copy(rows_v, out_hbm.at[pl.ds(base, _ROWS_PER_WORKER)])


def _sc_gather(codebook, idx_flat):
    mesh = plsc.VectorSubcoreMesh(core_axis_name="c", subcore_axis_name="s")
    run = functools.partial(
        pl.kernel,
        mesh=mesh,
        out_type=jax.ShapeDtypeStruct((ROWS, EMBED), jnp.float32),
        scratch_types=[
            pltpu.VMEM((_ROWS_PER_WORKER,), jnp.int32),
            pltpu.VMEM((_ROWS_PER_WORKER, EMBED), jnp.float32),
            pltpu.SemaphoreType.DMA,
        ],
    )(_sc_gather_body)
    return run(codebook, idx_flat)


def _output_body(z_ref, zq_ref, out_ref, loss_ref):
    i = pl.program_id(0)
    zt = z_ref[...]
    zn = _normalize(zt, axis=1)
    qn = _normalize(zq_ref[...], axis=1)
    out_ref[...] = zt + (qn - zt)
    diff = zn - qn
    part = jnp.sum(diff * diff)

    @pl.when(i == 0)
    def _init():
        loss_ref[...] = jnp.zeros_like(loss_ref)

    loss_ref[...] = loss_ref[...] + part

    @pl.when(i == _GRID2 - 1)
    def _fin():
        loss_ref[...] = loss_ref[...] * (1.25 / (ROWS * EMBED))


def _output_and_loss(z_flat, zq_raw):
    return pl.pallas_call(
        _output_body,
        grid=(_GRID2,),
        in_specs=[
            pl.BlockSpec((_TM2, EMBED), lambda i: (i, 0)),
            pl.BlockSpec((_TM2, EMBED), lambda i: (i, 0)),
        ],
        out_specs=[
            pl.BlockSpec((_TM2, EMBED), lambda i: (i, 0)),
            pl.BlockSpec((1, 1), lambda i: (0, 0)),
        ],
        out_shape=[
            jax.ShapeDtypeStruct((ROWS, EMBED), jnp.float32),
            jax.ShapeDtypeStruct((1, 1), jnp.float32),
        ],
    )(z_flat, zq_raw)


def kernel(z, codebook):
    z_flat = z.reshape(ROWS, EMBED)
    cb_t = codebook.T
    idx = _distance_indices(z_flat, cb_t).reshape(ROWS)
    zq_raw = _sc_gather(codebook, idx)
    z_q_st, loss = _output_and_loss(z_flat, zq_raw)
    return (z_q_st.reshape(z.shape), loss.reshape(()),
            idx.reshape(z.shape[:-1]))


# speedup vs baseline: 1.2536x; 1.1067x over previous
import functools

import jax
import jax.numpy as jnp
from jax import lax
from jax.experimental import pallas as pl
from jax.experimental.pallas import tpu as pltpu
from jax.experimental.pallas import tpu_sc as plsc

EMBED = 256
NCODES = 8192
ROWS = 8 * 576

_SC_CORES = 2
_SC_SUBCORES = 16
_SC_WORKERS = _SC_CORES * _SC_SUBCORES
_ROWS_PER_WORKER = ROWS // _SC_WORKERS

_TM = 256
_GRID = ROWS // _TM
_TM2 = 512
_GRID2 = ROWS // _TM2
_EPS = 1e-07


def _normalize(x, axis):
    n = jnp.sqrt(jnp.sum(x * x, axis=axis, keepdims=True))
    return x / jnp.maximum(n, _EPS)


def _distance_body(z_ref, cb_ref, idx_ref, et_ref, ne_ref):
    i = pl.program_id(0)

    @pl.when(i == 0)
    def _init():
        cbt = cb_ref[...].T
        et = _normalize(cbt, axis=0)
        et_ref[...] = et.astype(jnp.bfloat16)
        ne_ref[...] = jnp.sum(et * et, axis=0, keepdims=True)

    zt = z_ref[...]
    zn = _normalize(zt, axis=1)
    sumz = jnp.sum(zn * zn, axis=1, keepdims=True)
    lhs = (2.0 * zn).astype(jnp.bfloat16)
    s2 = jnp.dot(lhs, et_ref[...],
                 preferred_element_type=jnp.float32)
    d = (sumz - s2) + ne_ref[...]
    m = jnp.min(d, axis=1, keepdims=True)
    col = lax.broadcasted_iota(jnp.int32, d.shape, 1)
    idx = jnp.min(jnp.where(d == m, col, jnp.int32(NCODES)), axis=1)
    idx_ref[0, 0, :] = idx


def _distance_indices(z_flat, codebook):
    return pl.pallas_call(
        _distance_body,
        grid=(_GRID,),
        in_specs=[
            pl.BlockSpec((_TM, EMBED), lambda i: (i, 0)),
            pl.BlockSpec((NCODES, EMBED), lambda i: (0, 0)),
        ],
        out_specs=pl.BlockSpec((1, 1, _TM), lambda i: (i, 0, 0)),
        out_shape=jax.ShapeDtypeStruct((_GRID, 1, _TM), jnp.int32),
        scratch_shapes=[
            pltpu.VMEM((EMBED, NCODES), jnp.bfloat16),
            pltpu.VMEM((1, NCODES), jnp.float32),
        ],
    )(z_flat, codebook)


def _sc_gather_body(cb_hbm, idx_hbm, out_hbm, idx_v, rows_v, sem):
    wid = lax.axis_index("s") * _SC_CORES + lax.axis_index("c")
    base = wid * _ROWS_PER_WORKER
    pltpu.sync_copy(idx_hbm.at[pl.ds(base, _ROWS_PER_WORKER)], idx_v)
    pltpu.async_copy(cb_hbm.at[idx_v], rows_v, sem).wait()
    pltpu.sync_copy(rows_v, out_hbm.at[pl.ds(base, _ROWS_PER_WORKER)])


def _sc_gather(codebook, idx_flat):
    mesh = plsc.VectorSubcoreMesh(core_axis_name="c", subcore_axis_name="s")
    run = functools.partial(
        pl.kernel,
        mesh=mesh,
        out_type=jax.ShapeDtypeStruct((ROWS, EMBED), jnp.float32),
        scratch_types=[
            pltpu.VMEM((_ROWS_PER_WORKER,), jnp.int32),
            pltpu.VMEM((_ROWS_PER_WORKER, EMBED), jnp.float32),
            pltpu.SemaphoreType.DMA,
        ],
    )(_sc_gather_body)
    return run(codebook, idx_flat)


def _output_body(z_ref, zq_ref, out_ref, loss_ref):
    i = pl.program_id(0)
    zt = z_ref[...]
    zn = _normalize(zt, axis=1)
    qn = _normalize(zq_ref[...], axis=1)
    out_ref[...] = zt + (qn - zt)
    diff = zn - qn
    part = jnp.sum(diff * diff)

    @pl.when(i == 0)
    def _init():
        loss_ref[...] = jnp.zeros_like(loss_ref)

    loss_ref[...] = loss_ref[...] + part

    @pl.when(i == _GRID2 - 1)
    def _fin():
        loss_ref[...] = loss_ref[...] * (1.25 / (ROWS * EMBED))


def _output_and_loss(z_flat, zq_raw):
    return pl.pallas_call(
        _output_body,
        grid=(_GRID2,),
        in_specs=[
            pl.BlockSpec((_TM2, EMBED), lambda i: (i, 0)),
            pl.BlockSpec((_TM2, EMBED), lambda i: (i, 0)),
        ],
        out_specs=[
            pl.BlockSpec((_TM2, EMBED), lambda i: (i, 0)),
            pl.BlockSpec((1, 1), lambda i: (0, 0)),
        ],
        out_shape=[
            jax.ShapeDtypeStruct((ROWS, EMBED), jnp.float32),
            jax.ShapeDtypeStruct((1, 1), jnp.float32),
        ],
    )(z_flat, zq_raw)


def kernel(z, codebook):
    z_flat = z.reshape(ROWS, EMBED)
    idx = _distance_indices(z_flat, codebook).reshape(ROWS)
    zq_raw = _sc_gather(codebook, idx)
    z_q_st, loss = _output_and_loss(z_flat, zq_raw)
    return (z_q_st.reshape(z.shape), loss.reshape(()),
            idx.reshape(z.shape[:-1]))


# speedup vs baseline: 1.2769x; 1.0186x over previous
import functools

import jax
import jax.numpy as jnp
from jax import lax
from jax.experimental import pallas as pl
from jax.experimental.pallas import tpu as pltpu
from jax.experimental.pallas import tpu_sc as plsc

EMBED = 256
NCODES = 8192
ROWS = 8 * 576

_SC_CORES = 2
_SC_SUBCORES = 16
_SC_WORKERS = _SC_CORES * _SC_SUBCORES
_ROWS_PER_WORKER = ROWS // _SC_WORKERS

_TM = 512
_GRID = ROWS // _TM
_TM2 = 512
_GRID2 = ROWS // _TM2
_EPS = 1e-07


def _normalize(x, axis):
    n = jnp.sqrt(jnp.sum(x * x, axis=axis, keepdims=True))
    return x / jnp.maximum(n, _EPS)


def _distance_body(z_ref, cb_ref, idx_ref, et_ref, ne_ref):
    i = pl.program_id(0)

    @pl.when(i == 0)
    def _init():
        cbt = cb_ref[...].T
        et = _normalize(cbt, axis=0)
        et_ref[...] = et.astype(jnp.bfloat16)
        ne_ref[...] = jnp.sum(et * et, axis=0, keepdims=True)

    zt = z_ref[...]
    zn = _normalize(zt, axis=1)
    sumz = jnp.sum(zn * zn, axis=1, keepdims=True)
    lhs = (2.0 * zn).astype(jnp.bfloat16)
    s2 = jnp.dot(lhs, et_ref[...],
                 preferred_element_type=jnp.float32)
    d = (sumz - s2) + ne_ref[...]
    m = jnp.min(d, axis=1, keepdims=True)
    col = lax.broadcasted_iota(jnp.int32, (1, NCODES), 1)
    idx = jnp.min(jnp.where(d == m, col, jnp.int32(NCODES)), axis=1)
    idx_ref[0, 0, :] = idx


def _distance_indices(z_flat, codebook):
    return pl.pallas_call(
        _distance_body,
        grid=(_GRID,),
        in_specs=[
            pl.BlockSpec((_TM, EMBED), lambda i: (i, 0)),
            pl.BlockSpec((NCODES, EMBED), lambda i: (0, 0)),
        ],
        out_specs=pl.BlockSpec((1, 1, _TM), lambda i: (i, 0, 0)),
        out_shape=jax.ShapeDtypeStruct((_GRID, 1, _TM), jnp.int32),
        scratch_shapes=[
            pltpu.VMEM((EMBED, NCODES), jnp.bfloat16),
            pltpu.VMEM((1, NCODES), jnp.float32),
        ],
    )(z_flat, codebook)


def _sc_gather_body(cb_hbm, idx_hbm, out_hbm, idx_v, rows_v, sem):
    wid = lax.axis_index("s") * _SC_CORES + lax.axis_index("c")
    base = wid * _ROWS_PER_WORKER
    pltpu.sync_copy(idx_hbm.at[pl.ds(base, _ROWS_PER_WORKER)], idx_v)
    pltpu.async_copy(cb_hbm.at[idx_v], rows_v, sem).wait()
    pltpu.sync_copy(rows_v, out_hbm.at[pl.ds(base, _ROWS_PER_WORKER)])


def _sc_gather(codebook, idx_flat):
    mesh = plsc.VectorSubcoreMesh(core_axis_name="c", subcore_axis_name="s")
    run = functools.partial(
        pl.kernel,
        mesh=mesh,
        out_type=jax.ShapeDtypeStruct((ROWS, EMBED), jnp.float32),
        scratch_types=[
            pltpu.VMEM((_ROWS_PER_WORKER,), jnp.int32),
            pltpu.VMEM((_ROWS_PER_WORKER, EMBED), jnp.float32),
            pltpu.SemaphoreType.DMA,
        ],
    )(_sc_gather_body)
    return run(codebook, idx_flat)


def _output_body(z_ref, zq_ref, out_ref, loss_ref):
    i = pl.program_id(0)
    zt = z_ref[...]
    zn = _normalize(zt, axis=1)
    qn = _normalize(zq_ref[...], axis=1)
    out_ref[...] = zt + (qn - zt)
    diff = zn - qn
    part = jnp.sum(diff * diff)

    @pl.when(i == 0)
    def _init():
        loss_ref[...] = jnp.zeros_like(loss_ref)

    loss_ref[...] = loss_ref[...] + part

    @pl.when(i == _GRID2 - 1)
    def _fin():
        loss_ref[...] = loss_ref[...] * (1.25 / (ROWS * EMBED))


def _output_and_loss(z_flat, zq_raw):
    return pl.pallas_call(
        _output_body,
        grid=(_GRID2,),
        in_specs=[
            pl.BlockSpec((_TM2, EMBED), lambda i: (i, 0)),
            pl.BlockSpec((_TM2, EMBED), lambda i: (i, 0)),
        ],
        out_specs=[
            pl.BlockSpec((_TM2, EMBED), lambda i: (i, 0)),
            pl.BlockSpec((1, 1), lambda i: (0, 0)),
        ],
        out_shape=[
            jax.ShapeDtypeStruct((ROWS, EMBED), jnp.float32),
            jax.ShapeDtypeStruct((1, 1), jnp.float32),
        ],
    )(z_flat, zq_raw)


def kernel(z, codebook):
    z_flat = z.reshape(ROWS, EMBED)
    idx = _distance_indices(z_flat, codebook).reshape(ROWS)
    zq_raw = _sc_gather(codebook, idx)
    z_q_st, loss = _output_and_loss(z_flat, zq_raw)
    return (z_q_st.reshape(z.shape), loss.reshape(()),
            idx.reshape(z.shape[:-1]))


# speedup vs baseline: 1.4719x; 1.1528x over previous
import functools

import jax
import jax.numpy as jnp
from jax import lax
from jax.experimental import pallas as pl
from jax.experimental.pallas import tpu as pltpu
from jax.experimental.pallas import tpu_sc as plsc

EMBED = 256
NCODES = 8192
ROWS = 8 * 576

_SC_CORES = 2
_SC_SUBCORES = 16
_SC_WORKERS = _SC_CORES * _SC_SUBCORES
_ROWS_PER_WORKER = ROWS // _SC_WORKERS

_TM = 512
_GRID = ROWS // _TM
_TM2 = 512
_GRID2 = ROWS // _TM2
_EPS = 1e-07


def _normalize(x, axis):
    n = jnp.sqrt(jnp.sum(x * x, axis=axis, keepdims=True))
    return x / jnp.maximum(n, _EPS)


def _distance_body(z_ref, cb_ref, idx_ref, et_ref, ne_ref):
    i = pl.program_id(0)

    @pl.when(i == 0)
    def _init():
        cbt = cb_ref[...].T
        et = _normalize(cbt, axis=0)
        et_ref[...] = et.astype(jnp.bfloat16)
        ne_ref[...] = jnp.sum(et * et, axis=0, keepdims=True)

    zt = z_ref[...]
    zn = _normalize(zt, axis=1)
    sumz = jnp.sum(zn * zn, axis=1, keepdims=True)
    lhs = (2.0 * zn).astype(jnp.bfloat16)
    s2 = jnp.dot(lhs, et_ref[...],
                 preferred_element_type=jnp.float32)
    d = (sumz - s2) + ne_ref[...]
    idx = jnp.argmin(d, axis=1).astype(jnp.int32)
    idx_ref[0, 0, :] = idx


def _distance_indices(z_flat, codebook):
    return pl.pallas_call(
        _distance_body,
        grid=(_GRID,),
        in_specs=[
            pl.BlockSpec((_TM, EMBED), lambda i: (i, 0)),
            pl.BlockSpec((NCODES, EMBED), lambda i: (0, 0)),
        ],
        out_specs=pl.BlockSpec((1, 1, _TM), lambda i: (i, 0, 0)),
        out_shape=jax.ShapeDtypeStruct((_GRID, 1, _TM), jnp.int32),
        scratch_shapes=[
            pltpu.VMEM((EMBED, NCODES), jnp.bfloat16),
            pltpu.VMEM((1, NCODES), jnp.float32),
        ],
    )(z_flat, codebook)


def _sc_gather_body(cb_hbm, idx_hbm, out_hbm, idx_v, rows_v, sem):
    wid = lax.axis_index("s") * _SC_CORES + lax.axis_index("c")
    base = wid * _ROWS_PER_WORKER
    pltpu.sync_copy(idx_hbm.at[pl.ds(base, _ROWS_PER_WORKER)], idx_v)
    pltpu.async_copy(cb_hbm.at[idx_v], rows_v, sem).wait()
    pltpu.sync_copy(rows_v, out_hbm.at[pl.ds(base, _ROWS_PER_WORKER)])


def _sc_gather(codebook, idx_flat):
    mesh = plsc.VectorSubcoreMesh(core_axis_name="c", subcore_axis_name="s")
    run = functools.partial(
        pl.kernel,
        mesh=mesh,
        out_type=jax.ShapeDtypeStruct((ROWS, EMBED), jnp.float32),
        scratch_types=[
            pltpu.VMEM((_ROWS_PER_WORKER,), jnp.int32),
            pltpu.VMEM((_ROWS_PER_WORKER, EMBED), jnp.float32),
            pltpu.SemaphoreType.DMA,
        ],
    )(_sc_gather_body)
    return run(codebook, idx_flat)


def _output_body(z_ref, zq_ref, out_ref, loss_ref):
    i = pl.program_id(0)
    zt = z_ref[...]
    zn = _normalize(zt, axis=1)
    qn = _normalize(zq_ref[...], axis=1)
    out_ref[...] = zt + (qn - zt)
    diff = zn - qn
    part = jnp.sum(diff * diff)

    @pl.when(i == 0)
    def _init():
        loss_ref[...] = jnp.zeros_like(loss_ref)

    loss_ref[...] = loss_ref[...] + part

    @pl.when(i == _GRID2 - 1)
    def _fin():
        loss_ref[...] = loss_ref[...] * (1.25 / (ROWS * EMBED))


def _output_and_loss(z_flat, zq_raw):
    return pl.pallas_call(
        _output_body,
        grid=(_GRID2,),
        in_specs=[
            pl.BlockSpec((_TM2, EMBED), lambda i: (i, 0)),
            pl.BlockSpec((_TM2, EMBED), lambda i: (i, 0)),
        ],
        out_specs=[
            pl.BlockSpec((_TM2, EMBED), lambda i: (i, 0)),
            pl.BlockSpec((1, 1), lambda i: (0, 0)),
        ],
        out_shape=[
            jax.ShapeDtypeStruct((ROWS, EMBED), jnp.float32),
            jax.ShapeDtypeStruct((1, 1), jnp.float32),
        ],
    )(z_flat, zq_raw)


def kernel(z, codebook):
    z_flat = z.reshape(ROWS, EMBED)
    idx = _distance_indices(z_flat, codebook).reshape(ROWS)
    zq_raw = _sc_gather(codebook, idx)
    z_q_st, loss = _output_and_loss(z_flat, zq_raw)
    return (z_q_st.reshape(z.shape), loss.reshape(()),
            idx.reshape(z.shape[:-1]))
